# Initial kernel scaffold; baseline (speedup 1.0000x reference)
import jax
import jax.numpy as jnp
from jax.experimental import pallas as pl


def kernel(node_embeddings, edge_embeddings, edge_index_list, ln_gamma, ln_beta, W1, b1, W2, b2, W3, b3):
    raise NotImplementedError("write your pallas kernel here")


# speedup vs baseline: 1.9114x; 1.9114x over previous
import functools

import jax
import jax.numpy as jnp
from jax.experimental import pallas as pl
from jax.experimental.pallas import tpu as pltpu
from jax.experimental.pallas import tpu_sc as plsc

_NC = 2
_NS = 16
_LANES = 16


def _silu(v):
    return v * jax.nn.sigmoid(v)


def _ln_body(x_ref, g_ref, b_ref, o_ref):
    x = x_ref[...]
    mu = jnp.mean(x, axis=1, keepdims=True)
    xc = x - mu
    var = jnp.mean(xc * xc, axis=1, keepdims=True)
    o_ref[...] = xc / jnp.sqrt(var + 1e-5) * g_ref[...] + b_ref[...]


def _mlp_body(s_ref, w1_ref, b1_ref, w2_ref, b2_ref, o_ref):
    h = _silu(s_ref[...])
    h = jnp.dot(h, w1_ref[...], preferred_element_type=jnp.float32) + b1_ref[...]
    h = _silu(h)
    o_ref[...] = (
        jnp.dot(h, w2_ref[...], preferred_element_type=jnp.float32) + b2_ref[...]
    )


def _out_body(x_ref, a_ref, w3_ref, b3_ref, o_ref):
    t = _silu(x_ref[...] + a_ref[0] + a_ref[1])
    o_ref[...] = (
        jnp.dot(t, w3_ref[...], preferred_element_type=jnp.float32) + b3_ref[...]
    )


def _layer_norm_tc(x, gamma, beta, block_n):
    n, d = x.shape
    grid = n // block_n
    return pl.pallas_call(
        _ln_body,
        grid=(grid,),
        in_specs=[
            pl.BlockSpec((block_n, d), lambda i: (i, 0)),
            pl.BlockSpec((1, d), lambda i: (0, 0)),
            pl.BlockSpec((1, d), lambda i: (0, 0)),
        ],
        out_specs=pl.BlockSpec((block_n, d), lambda i: (i, 0)),
        out_shape=jax.ShapeDtypeStruct((n, d), jnp.float32),
    )(x, gamma, beta)


def _mlp_tc(s, w1, b1, w2, b2, block_e):
    e, d = s.shape
    h = w1.shape[1]
    grid = e // block_e
    return pl.pallas_call(
        _mlp_body,
        grid=(grid,),
        in_specs=[
            pl.BlockSpec((block_e, d), lambda i: (i, 0)),
            pl.BlockSpec((d, h), lambda i: (0, 0)),
            pl.BlockSpec((1, h), lambda i: (0, 0)),
            pl.BlockSpec((h, d), lambda i: (0, 0)),
            pl.BlockSpec((1, d), lambda i: (0, 0)),
        ],
        out_specs=pl.BlockSpec((block_e, d), lambda i: (i, 0)),
        out_shape=jax.ShapeDtypeStruct((e, d), jnp.float32),
    )(s, w1, b1, w2, b2)


def _final_tc(x, agg, w3, b3, block_n):
    n, d = x.shape
    grid = n // block_n
    return pl.pallas_call(
        _out_body,
        grid=(grid,),
        in_specs=[
            pl.BlockSpec((block_n, d), lambda i: (i, 0)),
            pl.BlockSpec((2, block_n, d), lambda i: (0, i, 0)),
            pl.BlockSpec((d, d), lambda i: (0, 0)),
            pl.BlockSpec((1, d), lambda i: (0, 0)),
        ],
        out_specs=pl.BlockSpec((block_n, d), lambda i: (i, 0)),
        out_shape=jax.ShapeDtypeStruct((n, d), jnp.float32),
    )(x, agg, w3, b3)


def _sc_gather_sum(x, edge_emb, idx_c, idx_n, window):
    e, d = edge_emb.shape
    mesh = plsc.VectorSubcoreMesh(core_axis_name="core", subcore_axis_name="subcore")

    @functools.partial(
        pl.kernel,
        out_type=jax.ShapeDtypeStruct((e, d), jnp.float32),
        mesh=mesh,
        scratch_types=[
            pltpu.VMEM((window, d), jnp.float32),
        ],
    )
    def gather_kernel(x_hbm, edge_hbm, ic_hbm, in_hbm, s_hbm, ne_buf):
        def body(ic_blk, in_blk, e_blk, s_blk):
            pltpu.sync_copy(x_hbm.at[ic_blk.at[0]], s_blk)
            pltpu.sync_copy(x_hbm.at[in_blk.at[0]], ne_buf)

            @pl.loop(0, window)
            def _row(r):
                @pl.loop(0, d, step=_LANES)
                def _col(c):
                    slc = (pl.ds(r, 1), pl.ds(c, _LANES))
                    s_blk.at[*slc][...] = (
                        s_blk.at[*slc][...]
                        + e_blk.at[*slc][...]
                        + ne_buf.at[*slc][...]
                    )

        pltpu.emit_pipeline(
            body,
            grid=(e // window,),
            in_specs=[
                pl.BlockSpec((1, window), lambda i: (0, i)),
                pl.BlockSpec((1, window), lambda i: (0, i)),
                pl.BlockSpec((window, d), lambda i: (i, 0)),
            ],
            out_specs=[pl.BlockSpec((window, d), lambda i: (i, 0))],
            core_axis_name=("core", "subcore"),
            dimension_semantics=(pltpu.PARALLEL,),
        )(ic_hbm, in_hbm, edge_hbm, s_hbm)

    return gather_kernel(x, edge_emb, idx_c, idx_n)


def _sc_scatter_agg(x, theta, idx_c, idx_n, window):
    e, d = theta.shape
    n = x.shape[0]
    chunk_rows = 80
    num_chunks = n // chunk_rows
    chunks_per_tile = -(-num_chunks // _NS)
    mesh = plsc.VectorSubcoreMesh(core_axis_name="core", subcore_axis_name="subcore")

    num_windows = e // window
    num_workers = _NC * _NS
    wloops = -(-num_windows // num_workers)

    @functools.partial(
        pl.kernel,
        out_type=jax.ShapeDtypeStruct((_NC, n, d), jnp.float32),
        mesh=mesh,
        scratch_types=[
            pltpu.VMEM((window, d), jnp.float32),
            pltpu.VMEM((window, d), jnp.float32),
            pltpu.VMEM((1, window), jnp.int32),
            pltpu.VMEM((1, window), jnp.int32),
            pltpu.VMEM_SHARED((n, d), jnp.float32),
        ],
    )
    def scatter_kernel(x_hbm, th_hbm, ic_hbm, in_hbm, agg_hbm,
                       ne_buf, th_buf, icb, inb, shared):
        cid = jax.lax.axis_index("core")
        sid = jax.lax.axis_index("subcore")
        wid = sid * _NC + cid

        @pl.loop(0, chunk_rows)
        def _zr(r):
            @pl.loop(0, d, step=_LANES)
            def _zc(c):
                ne_buf.at[pl.ds(r, 1), pl.ds(c, _LANES)][...] = jnp.zeros(
                    (1, _LANES), jnp.float32
                )

        @pl.loop(0, chunks_per_tile)
        def _z(k):
            j = sid + k * _NS

            @pl.when(j < num_chunks)
            def _():
                pltpu.sync_copy(
                    ne_buf.at[pl.ds(0, chunk_rows)],
                    shared.at[pl.ds(j * chunk_rows, chunk_rows)],
                )

        plsc.subcore_barrier()

        @pl.loop(0, wloops)
        def _w(k):
            w = wid + k * num_workers

            @pl.when(w < num_windows)
            def _():
                base = w * window
                pltpu.sync_copy(ic_hbm.at[:, pl.ds(base, window)], icb)
                pltpu.sync_copy(in_hbm.at[:, pl.ds(base, window)], inb)
                pltpu.sync_copy(th_hbm.at[pl.ds(base, window)], th_buf)
                pltpu.sync_copy(x_hbm.at[inb.at[0]], ne_buf)

                @pl.loop(0, window)
                def _row(r):
                    @pl.loop(0, d, step=_LANES)
                    def _col(c):
                        slc = (pl.ds(r, 1), pl.ds(c, _LANES))
                        ne_buf.at[*slc][...] = (
                            ne_buf.at[*slc][...] * th_buf.at[*slc][...]
                        )

                pltpu.sync_copy(ne_buf, shared.at[icb.at[0]], add=True)

        plsc.subcore_barrier()

        @pl.loop(0, chunks_per_tile)
        def _d(k):
            j = sid + k * _NS

            @pl.when(j < num_chunks)
            def _():
                off = j * chunk_rows
                pltpu.sync_copy(shared.at[pl.ds(off, chunk_rows)],
                                th_buf.at[pl.ds(0, chunk_rows)])
                pltpu.sync_copy(th_buf.at[pl.ds(0, chunk_rows)],
                                agg_hbm.at[cid, pl.ds(off, chunk_rows)])

    return scatter_kernel(x, theta, idx_c, idx_n)


def kernel(node_embeddings, edge_embeddings, edge_index_list,
           ln_gamma, ln_beta, W1, b1, W2, b2, W3, b3):
    n, d = node_embeddings.shape
    e = edge_embeddings.shape[0]
    h = W1.shape[1]

    idx = edge_index_list.astype(jnp.int32)
    idx_c = idx[0:1, :]
    idx_n = idx[1:2, :]

    window = 128

    x = _layer_norm_tc(
        node_embeddings, ln_gamma.reshape(1, d), ln_beta.reshape(1, d), block_n=400
    )
    s = _sc_gather_sum(x, edge_embeddings, idx_c, idx_n, window)
    theta = _mlp_tc(s, W1, b1.reshape(1, h), W2, b2.reshape(1, d), block_e=3200)
    agg = _sc_scatter_agg(x, theta, idx_c, idx_n, window)
    out = _final_tc(x, agg, W3, b3.reshape(1, d), block_n=400)
    return out


# speedup vs baseline: 3.1998x; 1.6741x over previous
import functools

import jax
import jax.numpy as jnp
from jax.experimental import pallas as pl
from jax.experimental.pallas import tpu as pltpu
from jax.experimental.pallas import tpu_sc as plsc

_NC = 2
_NS = 16
_LANES = 16


def _silu(v):
    return v * jax.nn.sigmoid(v)


def _ln_body(x_ref, g_ref, b_ref, o_ref):
    x = x_ref[...]
    mu = jnp.mean(x, axis=1, keepdims=True)
    xc = x - mu
    var = jnp.mean(xc * xc, axis=1, keepdims=True)
    o_ref[...] = xc / jnp.sqrt(var + 1e-5) * g_ref[...] + b_ref[...]


def _mlp_body(e_ref, s_ref, w1_ref, b1_ref, w2_ref, b2_ref, o_ref):
    h = _silu(e_ref[...] + s_ref[...])
    h = jnp.dot(h, w1_ref[...], preferred_element_type=jnp.float32) + b1_ref[...]
    h = _silu(h)
    o_ref[...] = (
        jnp.dot(h, w2_ref[...], preferred_element_type=jnp.float32) + b2_ref[...]
    )


def _out_body(x_ref, a_ref, w3_ref, b3_ref, o_ref):
    t = _silu(x_ref[...] + a_ref[0] + a_ref[1])
    o_ref[...] = (
        jnp.dot(t, w3_ref[...], preferred_element_type=jnp.float32) + b3_ref[...]
    )


def _layer_norm_tc(x, gamma, beta, block_n):
    n, d = x.shape
    grid = n // block_n
    return pl.pallas_call(
        _ln_body,
        grid=(grid,),
        in_specs=[
            pl.BlockSpec((block_n, d), lambda i: (i, 0)),
            pl.BlockSpec((1, d), lambda i: (0, 0)),
            pl.BlockSpec((1, d), lambda i: (0, 0)),
        ],
        out_specs=pl.BlockSpec((block_n, d), lambda i: (i, 0)),
        out_shape=jax.ShapeDtypeStruct((n, d), jnp.float32),
    )(x, gamma, beta)


def _mlp_tc(edge_emb, s, w1, b1, w2, b2, block_e):
    e, d = s.shape
    h = w1.shape[1]
    grid = e // block_e
    return pl.pallas_call(
        _mlp_body,
        grid=(grid,),
        in_specs=[
            pl.BlockSpec((block_e, d), lambda i: (i, 0)),
            pl.BlockSpec((block_e, d), lambda i: (i, 0)),
            pl.BlockSpec((d, h), lambda i: (0, 0)),
            pl.BlockSpec((1, h), lambda i: (0, 0)),
            pl.BlockSpec((h, d), lambda i: (0, 0)),
            pl.BlockSpec((1, d), lambda i: (0, 0)),
        ],
        out_specs=pl.BlockSpec((block_e, d), lambda i: (i, 0)),
        out_shape=jax.ShapeDtypeStruct((e, d), jnp.float32),
    )(edge_emb, s, w1, b1, w2, b2)


def _final_tc(x, agg, w3, b3, block_n):
    n, d = x.shape
    grid = n // block_n
    return pl.pallas_call(
        _out_body,
        grid=(grid,),
        in_specs=[
            pl.BlockSpec((block_n, d), lambda i: (i, 0)),
            pl.BlockSpec((2, block_n, d), lambda i: (0, i, 0)),
            pl.BlockSpec((d, d), lambda i: (0, 0)),
            pl.BlockSpec((1, d), lambda i: (0, 0)),
        ],
        out_specs=pl.BlockSpec((block_n, d), lambda i: (i, 0)),
        out_shape=jax.ShapeDtypeStruct((n, d), jnp.float32),
    )(x, agg, w3, b3)


def _sc_gather_sum(x, idx_c, idx_n, e, window):
    d = x.shape[1]
    n_win = e // window
    num_workers = _NC * _NS
    slots = ((-(-n_win // num_workers) + 2) // 2) * 2
    mesh = plsc.VectorSubcoreMesh(core_axis_name="core", subcore_axis_name="subcore")

    @functools.partial(
        pl.kernel,
        out_type=jax.ShapeDtypeStruct((e, d), jnp.float32),
        mesh=mesh,
        scratch_types=[
            pltpu.VMEM((window, d), jnp.float32),
            pltpu.VMEM((window, d), jnp.float32),
            pltpu.VMEM((window, d), jnp.float32),
            pltpu.VMEM((window, d), jnp.float32),
            pltpu.VMEM((1, window), jnp.int32),
            pltpu.VMEM((1, window), jnp.int32),
            pltpu.VMEM((1, window), jnp.int32),
            pltpu.VMEM((1, window), jnp.int32),
            pltpu.SemaphoreType.DMA,
            pltpu.SemaphoreType.DMA,
            pltpu.SemaphoreType.DMA,
            pltpu.SemaphoreType.DMA,
            pltpu.SemaphoreType.DMA,
            pltpu.SemaphoreType.DMA,
        ],
    )
    def gather_kernel(x_hbm, ic_hbm, in_hbm, s_hbm,
                      ce0, ce1, ne0, ne1, ic0, ic1, in0, in1,
                      si0, si1, sg0, sg1, so0, so1):
        cid = jax.lax.axis_index("core")
        sid = jax.lax.axis_index("subcore")
        wid = sid * _NC + cid
        ce = (ce0, ce1)
        ne = (ne0, ne1)
        icb = (ic0, ic1)
        inb = (in0, in1)
        si = (si0, si1)
        sg = (sg0, sg1)
        so = (so0, so1)

        pltpu.async_copy(ic_hbm.at[:, pl.ds(wid * window, window)], icb[0], si[0])
        pltpu.async_copy(in_hbm.at[:, pl.ds(wid * window, window)], inb[0], si[0])

        @pl.loop(0, slots // 2)
        def _outer(j):
            for p in range(2):
                k = j * 2 + p
                q = 1 - p
                w = wid + k * num_workers
                w_prev = w - num_workers
                w_next = w + num_workers

                @pl.when(w < n_win)
                def _():
                    @pl.when(k >= 2)
                    def _():
                        pltpu.make_async_copy(
                            ce[p], s_hbm.at[pl.ds(0, window)], so[p]
                        ).wait()

                    pltpu.make_async_copy(
                        ic_hbm.at[:, pl.ds(0, window)], icb[p], si[p]
                    ).wait()
                    pltpu.make_async_copy(
                        in_hbm.at[:, pl.ds(0, window)], inb[p], si[p]
                    ).wait()
                    pltpu.async_copy(x_hbm.at[icb[p].at[0]], ce[p], sg[p])
                    pltpu.async_copy(x_hbm.at[inb[p].at[0]], ne[p], sg[p])

                @pl.when((k >= 1) & (w_prev < n_win))
                def _():
                    pltpu.make_async_copy(
                        x_hbm.at[pl.ds(0, window)], ce[q], sg[q]
                    ).wait()
                    pltpu.make_async_copy(
                        x_hbm.at[pl.ds(0, window)], ne[q], sg[q]
                    ).wait()

                @pl.when(w_next < n_win)
                def _():
                    pltpu.async_copy(
                        ic_hbm.at[:, pl.ds(w_next * window, window)], icb[q], si[q]
                    )
                    pltpu.async_copy(
                        in_hbm.at[:, pl.ds(w_next * window, window)], inb[q], si[q]
                    )

                @pl.when((k >= 1) & (w_prev < n_win))
                def _():
                    @pl.loop(0, window)
                    def _row(r):
                        @pl.loop(0, d, step=_LANES)
                        def _col(c):
                            slc = (pl.ds(r, 1), pl.ds(c, _LANES))
                            ce[q].at[*slc][...] = (
                                ce[q].at[*slc][...] + ne[q].at[*slc][...]
                            )

                    pltpu.async_copy(
                        ce[q], s_hbm.at[pl.ds(w_prev * window, window)], so[q]
                    )

        pltpu.make_async_copy(ce0, s_hbm.at[pl.ds(0, window)], so0).wait()
        pltpu.make_async_copy(ce1, s_hbm.at[pl.ds(0, window)], so1).wait()

    return gather_kernel(x, idx_c, idx_n)


def _sc_scatter_agg(x, theta, idx_c, idx_n, window):
    e, d = theta.shape
    n = x.shape[0]
    chunk_rows = 80
    num_chunks = n // chunk_rows
    chunks_per_tile = -(-num_chunks // _NS)
    mesh = plsc.VectorSubcoreMesh(core_axis_name="core", subcore_axis_name="subcore")

    num_windows = e // window
    num_workers = _NC * _NS
    wloops = -(-num_windows // num_workers)

    @functools.partial(
        pl.kernel,
        out_type=jax.ShapeDtypeStruct((_NC, n, d), jnp.float32),
        mesh=mesh,
        scratch_types=[
            pltpu.VMEM((window, d), jnp.float32),
            pltpu.VMEM((window, d), jnp.float32),
            pltpu.VMEM((1, window), jnp.int32),
            pltpu.VMEM((1, window), jnp.int32),
            pltpu.VMEM_SHARED((n, d), jnp.float32),
        ],
    )
    def scatter_kernel(x_hbm, th_hbm, ic_hbm, in_hbm, agg_hbm,
                       ne_buf, th_buf, icb, inb, shared):
        cid = jax.lax.axis_index("core")
        sid = jax.lax.axis_index("subcore")
        wid = sid * _NC + cid

        @pl.loop(0, chunk_rows)
        def _zr(r):
            @pl.loop(0, d, step=_LANES)
            def _zc(c):
                ne_buf.at[pl.ds(r, 1), pl.ds(c, _LANES)][...] = jnp.zeros(
                    (1, _LANES), jnp.float32
                )

        @pl.loop(0, chunks_per_tile)
        def _z(k):
            j = sid + k * _NS

            @pl.when(j < num_chunks)
            def _():
                pltpu.sync_copy(
                    ne_buf.at[pl.ds(0, chunk_rows)],
                    shared.at[pl.ds(j * chunk_rows, chunk_rows)],
                )

        plsc.subcore_barrier()

        @pl.loop(0, wloops)
        def _w(k):
            w = wid + k * num_workers

            @pl.when(w < num_windows)
            def _():
                base = w * window
                pltpu.sync_copy(ic_hbm.at[:, pl.ds(base, window)], icb)
                pltpu.sync_copy(in_hbm.at[:, pl.ds(base, window)], inb)
                pltpu.sync_copy(th_hbm.at[pl.ds(base, window)], th_buf)
                pltpu.sync_copy(x_hbm.at[inb.at[0]], ne_buf)

                @pl.loop(0, window)
                def _row(r):
                    @pl.loop(0, d, step=_LANES)
                    def _col(c):
                        slc = (pl.ds(r, 1), pl.ds(c, _LANES))
                        ne_buf.at[*slc][...] = (
                            ne_buf.at[*slc][...] * th_buf.at[*slc][...]
                        )

                pltpu.sync_copy(ne_buf, shared.at[icb.at[0]], add=True)

        plsc.subcore_barrier()

        @pl.loop(0, chunks_per_tile)
        def _d(k):
            j = sid + k * _NS

            @pl.when(j < num_chunks)
            def _():
                off = j * chunk_rows
                pltpu.sync_copy(shared.at[pl.ds(off, chunk_rows)],
                                th_buf.at[pl.ds(0, chunk_rows)])
                pltpu.sync_copy(th_buf.at[pl.ds(0, chunk_rows)],
                                agg_hbm.at[cid, pl.ds(off, chunk_rows)])

    return scatter_kernel(x, theta, idx_c, idx_n)


def kernel(node_embeddings, edge_embeddings, edge_index_list,
           ln_gamma, ln_beta, W1, b1, W2, b2, W3, b3):
    n, d = node_embeddings.shape
    e = edge_embeddings.shape[0]
    h = W1.shape[1]

    idx = edge_index_list.astype(jnp.int32)
    idx_c = idx[0:1, :]
    idx_n = idx[1:2, :]

    window = 128

    x = _layer_norm_tc(
        node_embeddings, ln_gamma.reshape(1, d), ln_beta.reshape(1, d), block_n=400
    )
    s = _sc_gather_sum(x, idx_c, idx_n, e, window)
    theta = _mlp_tc(edge_embeddings, s, W1, b1.reshape(1, h), W2, b2.reshape(1, d),
                    block_e=3200)
    agg = _sc_scatter_agg(x, theta, idx_c, idx_n, window)
    out = _final_tc(x, agg, W3, b3.reshape(1, d), block_n=400)
    return out


# speedup vs baseline: 4.2520x; 1.3288x over previous
import functools

import jax
import jax.numpy as jnp
from jax.experimental import pallas as pl
from jax.experimental.pallas import tpu as pltpu
from jax.experimental.pallas import tpu_sc as plsc

_NC = 2
_NS = 16
_LANES = 16


def _silu(v):
    return v * jax.nn.sigmoid(v)


def _ln_body(x_ref, g_ref, b_ref, o_ref):
    x = x_ref[...]
    mu = jnp.mean(x, axis=1, keepdims=True)
    xc = x - mu
    var = jnp.mean(xc * xc, axis=1, keepdims=True)
    o_ref[...] = xc / jnp.sqrt(var + 1e-5) * g_ref[...] + b_ref[...]


def _mlp_body(e_ref, s_ref, w1_ref, b1_ref, w2_ref, b2_ref, o_ref):
    h = _silu(e_ref[...] + s_ref[...])
    h = jnp.dot(h, w1_ref[...], preferred_element_type=jnp.float32) + b1_ref[...]
    h = _silu(h)
    o_ref[...] = (
        jnp.dot(h, w2_ref[...], preferred_element_type=jnp.float32) + b2_ref[...]
    )


def _out_body(x_ref, a_ref, w3_ref, b3_ref, o_ref):
    t = _silu(x_ref[...] + a_ref[0] + a_ref[1])
    o_ref[...] = (
        jnp.dot(t, w3_ref[...], preferred_element_type=jnp.float32) + b3_ref[...]
    )


def _layer_norm_tc(x, gamma, beta, block_n):
    n, d = x.shape
    grid = n // block_n
    return pl.pallas_call(
        _ln_body,
        grid=(grid,),
        in_specs=[
            pl.BlockSpec((block_n, d), lambda i: (i, 0)),
            pl.BlockSpec((1, d), lambda i: (0, 0)),
            pl.BlockSpec((1, d), lambda i: (0, 0)),
        ],
        out_specs=pl.BlockSpec((block_n, d), lambda i: (i, 0)),
        out_shape=jax.ShapeDtypeStruct((n, d), jnp.float32),
    )(x, gamma, beta)


def _mlp_tc(edge_emb, s, w1, b1, w2, b2, block_e):
    e, d = s.shape
    h = w1.shape[1]
    grid = e // block_e
    return pl.pallas_call(
        _mlp_body,
        grid=(grid,),
        in_specs=[
            pl.BlockSpec((block_e, d), lambda i: (i, 0)),
            pl.BlockSpec((block_e, d), lambda i: (i, 0)),
            pl.BlockSpec((d, h), lambda i: (0, 0)),
            pl.BlockSpec((1, h), lambda i: (0, 0)),
            pl.BlockSpec((h, d), lambda i: (0, 0)),
            pl.BlockSpec((1, d), lambda i: (0, 0)),
        ],
        out_specs=pl.BlockSpec((block_e, d), lambda i: (i, 0)),
        out_shape=jax.ShapeDtypeStruct((e, d), jnp.float32),
    )(edge_emb, s, w1, b1, w2, b2)


def _final_tc(x, agg, w3, b3, block_n):
    n, d = x.shape
    grid = n // block_n
    return pl.pallas_call(
        _out_body,
        grid=(grid,),
        in_specs=[
            pl.BlockSpec((block_n, d), lambda i: (i, 0)),
            pl.BlockSpec((2, block_n, d), lambda i: (0, i, 0)),
            pl.BlockSpec((d, d), lambda i: (0, 0)),
            pl.BlockSpec((1, d), lambda i: (0, 0)),
        ],
        out_specs=pl.BlockSpec((block_n, d), lambda i: (i, 0)),
        out_shape=jax.ShapeDtypeStruct((n, d), jnp.float32),
    )(x, agg, w3, b3)


def _sc_gather_sum(x, idx_c, idx_n, e, window):
    d = x.shape[1]
    n_win = e // window
    num_workers = _NC * _NS
    slots = ((-(-n_win // num_workers) + 2) // 2) * 2
    mesh = plsc.VectorSubcoreMesh(core_axis_name="core", subcore_axis_name="subcore")

    @functools.partial(
        pl.kernel,
        out_type=jax.ShapeDtypeStruct((e, d), jnp.float32),
        mesh=mesh,
        scratch_types=[
            pltpu.VMEM((window, d), jnp.float32),
            pltpu.VMEM((window, d), jnp.float32),
            pltpu.VMEM((window, d), jnp.float32),
            pltpu.VMEM((window, d), jnp.float32),
            pltpu.VMEM((1, window), jnp.int32),
            pltpu.VMEM((1, window), jnp.int32),
            pltpu.VMEM((1, window), jnp.int32),
            pltpu.VMEM((1, window), jnp.int32),
            pltpu.SemaphoreType.DMA,
            pltpu.SemaphoreType.DMA,
            pltpu.SemaphoreType.DMA,
            pltpu.SemaphoreType.DMA,
            pltpu.SemaphoreType.DMA,
            pltpu.SemaphoreType.DMA,
        ],
    )
    def gather_kernel(x_hbm, ic_hbm, in_hbm, s_hbm,
                      ce0, ce1, ne0, ne1, ic0, ic1, in0, in1,
                      si0, si1, sg0, sg1, so0, so1):
        cid = jax.lax.axis_index("core")
        sid = jax.lax.axis_index("subcore")
        wid = sid * _NC + cid
        ce = (ce0, ce1)
        ne = (ne0, ne1)
        icb = (ic0, ic1)
        inb = (in0, in1)
        si = (si0, si1)
        sg = (sg0, sg1)
        so = (so0, so1)

        pltpu.async_copy(ic_hbm.at[:, pl.ds(wid * window, window)], icb[0], si[0])
        pltpu.async_copy(in_hbm.at[:, pl.ds(wid * window, window)], inb[0], si[0])

        @pl.loop(0, slots // 2)
        def _outer(j):
            for p in range(2):
                k = j * 2 + p
                q = 1 - p
                w = wid + k * num_workers
                w_prev = w - num_workers
                w_next = w + num_workers

                @pl.when(w < n_win)
                def _():
                    @pl.when(k >= 2)
                    def _():
                        pltpu.make_async_copy(
                            ce[p], s_hbm.at[pl.ds(0, window)], so[p]
                        ).wait()

                    pltpu.make_async_copy(
                        ic_hbm.at[:, pl.ds(0, window)], icb[p], si[p]
                    ).wait()
                    pltpu.make_async_copy(
                        in_hbm.at[:, pl.ds(0, window)], inb[p], si[p]
                    ).wait()
                    pltpu.async_copy(x_hbm.at[icb[p].at[0]], ce[p], sg[p])
                    pltpu.async_copy(x_hbm.at[inb[p].at[0]], ne[p], sg[p])

                @pl.when((k >= 1) & (w_prev < n_win))
                def _():
                    pltpu.make_async_copy(
                        x_hbm.at[pl.ds(0, window)], ce[q], sg[q]
                    ).wait()
                    pltpu.make_async_copy(
                        x_hbm.at[pl.ds(0, window)], ne[q], sg[q]
                    ).wait()

                @pl.when(w_next < n_win)
                def _():
                    pltpu.async_copy(
                        ic_hbm.at[:, pl.ds(w_next * window, window)], icb[q], si[q]
                    )
                    pltpu.async_copy(
                        in_hbm.at[:, pl.ds(w_next * window, window)], inb[q], si[q]
                    )

                @pl.when((k >= 1) & (w_prev < n_win))
                def _():
                    @pl.loop(0, window)
                    def _row(r):
                        @pl.loop(0, d, step=_LANES)
                        def _col(c):
                            slc = (pl.ds(r, 1), pl.ds(c, _LANES))
                            ce[q].at[*slc][...] = (
                                ce[q].at[*slc][...] + ne[q].at[*slc][...]
                            )

                    pltpu.async_copy(
                        ce[q], s_hbm.at[pl.ds(w_prev * window, window)], so[q]
                    )

        pltpu.make_async_copy(ce0, s_hbm.at[pl.ds(0, window)], so0).wait()
        pltpu.make_async_copy(ce1, s_hbm.at[pl.ds(0, window)], so1).wait()

    return gather_kernel(x, idx_c, idx_n)


def _sc_scatter_agg(x, theta, idx_c, idx_n, window):
    e, d = theta.shape
    n = x.shape[0]
    chunk_rows = 80
    num_chunks = n // chunk_rows
    chunks_per_tile = -(-num_chunks // _NS)
    mesh = plsc.VectorSubcoreMesh(core_axis_name="core", subcore_axis_name="subcore")

    num_windows = e // window
    num_workers = _NC * _NS
    slots = ((-(-num_windows // num_workers) + 2) // 2) * 2

    @functools.partial(
        pl.kernel,
        out_type=jax.ShapeDtypeStruct((_NC, n, d), jnp.float32),
        mesh=mesh,
        scratch_types=[
            pltpu.VMEM((window, d), jnp.float32),
            pltpu.VMEM((window, d), jnp.float32),
            pltpu.VMEM((window, d), jnp.float32),
            pltpu.VMEM((1, window), jnp.int32),
            pltpu.VMEM((1, window), jnp.int32),
            pltpu.VMEM((1, window), jnp.int32),
            pltpu.VMEM((1, window), jnp.int32),
            pltpu.VMEM_SHARED((n, d), jnp.float32),
            pltpu.SemaphoreType.DMA,
            pltpu.SemaphoreType.DMA,
            pltpu.SemaphoreType.DMA,
            pltpu.SemaphoreType.DMA,
            pltpu.SemaphoreType.DMA,
            pltpu.SemaphoreType.DMA,
            pltpu.SemaphoreType.DMA,
            pltpu.SemaphoreType.DMA,
            pltpu.SemaphoreType.DMA,
        ],
    )
    def scatter_kernel(x_hbm, th_hbm, ic_hbm, in_hbm, agg_hbm,
                       ne0, ne1, th_buf, ic0, ic1, in0, in1, shared,
                       si0, si1, sic0, sic1, sg0, sg1, st, so0, so1):
        cid = jax.lax.axis_index("core")
        sid = jax.lax.axis_index("subcore")
        wid = sid * _NC + cid
        ne = (ne0, ne1)
        icb = (ic0, ic1)
        inb = (in0, in1)
        si = (si0, si1)
        sic = (sic0, sic1)
        sg = (sg0, sg1)
        so = (so0, so1)

        @pl.loop(0, chunk_rows)
        def _zr(r):
            @pl.loop(0, d, step=_LANES)
            def _zc(c):
                ne0.at[pl.ds(r, 1), pl.ds(c, _LANES)][...] = jnp.zeros(
                    (1, _LANES), jnp.float32
                )

        @pl.loop(0, chunks_per_tile)
        def _z(k):
            j = sid + k * _NS

            @pl.when(j < num_chunks)
            def _():
                pltpu.sync_copy(
                    ne0.at[pl.ds(0, chunk_rows)],
                    shared.at[pl.ds(j * chunk_rows, chunk_rows)],
                )

        plsc.subcore_barrier()

        pltpu.async_copy(in_hbm.at[:, pl.ds(wid * window, window)], inb[0], si[0])

        @pl.loop(0, slots // 2)
        def _outer(j):
            for p in range(2):
                k = j * 2 + p
                q = 1 - p
                w = wid + k * num_workers
                w_prev = w - num_workers
                w_next = w + num_workers
                base = w * window

                @pl.when(w < num_windows)
                def _():
                    @pl.when(k >= 2)
                    def _():
                        pltpu.make_async_copy(
                            ne[p], shared.at[pl.ds(0, window)], so[p]
                        ).wait()

                    pltpu.make_async_copy(
                        in_hbm.at[:, pl.ds(0, window)], inb[p], si[p]
                    ).wait()
                    pltpu.async_copy(x_hbm.at[inb[p].at[0]], ne[p], sg[p])

                @pl.when((k >= 1) & (w_prev < num_windows))
                def _():
                    pltpu.make_async_copy(
                        x_hbm.at[pl.ds(0, window)], ne[q], sg[q]
                    ).wait()

                @pl.when(w_next < num_windows)
                def _():
                    pltpu.async_copy(
                        in_hbm.at[:, pl.ds(w_next * window, window)], inb[q], si[q]
                    )

                @pl.when((k >= 1) & (w_prev < num_windows))
                def _():
                    pltpu.make_async_copy(
                        th_hbm.at[pl.ds(0, window)], th_buf, st
                    ).wait()

                    @pl.loop(0, window)
                    def _row(r):
                        @pl.loop(0, d, step=_LANES)
                        def _col(c):
                            slc = (pl.ds(r, 1), pl.ds(c, _LANES))
                            ne[q].at[*slc][...] = (
                                ne[q].at[*slc][...] * th_buf.at[*slc][...]
                            )

                @pl.when(w < num_windows)
                def _():
                    pltpu.async_copy(th_hbm.at[pl.ds(base, window)], th_buf, st)

                @pl.when((k >= 1) & (w_prev < num_windows))
                def _():
                    pltpu.make_async_copy(
                        ic_hbm.at[:, pl.ds(0, window)], icb[q], sic[q]
                    ).wait()
                    pltpu.async_copy(ne[q], shared.at[icb[q].at[0]], so[q], add=True)

                @pl.when(w < num_windows)
                def _():
                    pltpu.async_copy(
                        ic_hbm.at[:, pl.ds(base, window)], icb[p], sic[p]
                    )

        pltpu.make_async_copy(ne0, shared.at[pl.ds(0, window)], so0).wait()
        pltpu.make_async_copy(ne1, shared.at[pl.ds(0, window)], so1).wait()

        plsc.subcore_barrier()

        @pl.loop(0, chunks_per_tile)
        def _d(k):
            j = sid + k * _NS

            @pl.when(j < num_chunks)
            def _():
                off = j * chunk_rows
                pltpu.sync_copy(shared.at[pl.ds(off, chunk_rows)],
                                th_buf.at[pl.ds(0, chunk_rows)])
                pltpu.sync_copy(th_buf.at[pl.ds(0, chunk_rows)],
                                agg_hbm.at[cid, pl.ds(off, chunk_rows)])

    return scatter_kernel(x, theta, idx_c, idx_n)


def kernel(node_embeddings, edge_embeddings, edge_index_list,
           ln_gamma, ln_beta, W1, b1, W2, b2, W3, b3):
    n, d = node_embeddings.shape
    e = edge_embeddings.shape[0]
    h = W1.shape[1]

    idx = edge_index_list.astype(jnp.int32)
    idx_c = idx[0:1, :]
    idx_n = idx[1:2, :]

    window = 128

    x = _layer_norm_tc(
        node_embeddings, ln_gamma.reshape(1, d), ln_beta.reshape(1, d), block_n=400
    )
    s = _sc_gather_sum(x, idx_c, idx_n, e, window)
    theta = _mlp_tc(edge_embeddings, s, W1, b1.reshape(1, h), W2, b2.reshape(1, d),
                    block_e=3200)
    agg = _sc_scatter_agg(x, theta, idx_c, idx_n, window)
    out = _final_tc(x, agg, W3, b3.reshape(1, d), block_n=400)
    return out


# speedup vs baseline: 4.4248x; 1.0406x over previous
import functools

import jax
import jax.numpy as jnp
from jax.experimental import pallas as pl
from jax.experimental.pallas import tpu as pltpu
from jax.experimental.pallas import tpu_sc as plsc

_NC = 2
_NS = 16
_LANES = 16


def _silu(v):
    return v * jax.nn.sigmoid(v)


def _ln_body(x_ref, g_ref, b_ref, o_ref):
    x = x_ref[...]
    mu = jnp.mean(x, axis=1, keepdims=True)
    xc = x - mu
    var = jnp.mean(xc * xc, axis=1, keepdims=True)
    o_ref[...] = xc / jnp.sqrt(var + 1e-5) * g_ref[...] + b_ref[...]


def _mlp_body(e_ref, s_ref, w1_ref, b1_ref, w2_ref, b2_ref, o_ref):
    h = _silu(e_ref[...] + s_ref[...]).astype(jnp.bfloat16)
    h = jnp.dot(h, w1_ref[...], preferred_element_type=jnp.float32) + b1_ref[...]
    h = _silu(h).astype(jnp.bfloat16)
    o_ref[...] = (
        jnp.dot(h, w2_ref[...], preferred_element_type=jnp.float32) + b2_ref[...]
    )


def _out_body(x_ref, a_ref, w3_ref, b3_ref, o_ref):
    t = _silu(x_ref[...] + a_ref[0] + a_ref[1])
    o_ref[...] = (
        jnp.dot(t, w3_ref[...], preferred_element_type=jnp.float32) + b3_ref[...]
    )


def _layer_norm_tc(x, gamma, beta, block_n):
    n, d = x.shape
    grid = n // block_n
    return pl.pallas_call(
        _ln_body,
        grid=(grid,),
        in_specs=[
            pl.BlockSpec((block_n, d), lambda i: (i, 0)),
            pl.BlockSpec((1, d), lambda i: (0, 0)),
            pl.BlockSpec((1, d), lambda i: (0, 0)),
        ],
        out_specs=pl.BlockSpec((block_n, d), lambda i: (i, 0)),
        out_shape=jax.ShapeDtypeStruct((n, d), jnp.float32),
    )(x, gamma, beta)


def _mlp_tc(edge_emb, s, w1, b1, w2, b2, block_e, chunk_block0):
    ec, d = s.shape
    h = w1.shape[1]
    grid = ec // block_e
    return pl.pallas_call(
        _mlp_body,
        grid=(grid,),
        in_specs=[
            pl.BlockSpec((block_e, d), lambda i: (chunk_block0 + i, 0)),
            pl.BlockSpec((block_e, d), lambda i: (i, 0)),
            pl.BlockSpec((d, h), lambda i: (0, 0)),
            pl.BlockSpec((1, h), lambda i: (0, 0)),
            pl.BlockSpec((h, d), lambda i: (0, 0)),
            pl.BlockSpec((1, d), lambda i: (0, 0)),
        ],
        out_specs=pl.BlockSpec((block_e, d), lambda i: (i, 0)),
        out_shape=jax.ShapeDtypeStruct((ec, d), jnp.float32),
    )(edge_emb, s, w1, b1, w2, b2)


def _final_tc(x, agg, w3, b3, block_n):
    n, d = x.shape
    grid = n // block_n
    return pl.pallas_call(
        _out_body,
        grid=(grid,),
        in_specs=[
            pl.BlockSpec((block_n, d), lambda i: (i, 0)),
            pl.BlockSpec((2, block_n, d), lambda i: (0, i, 0)),
            pl.BlockSpec((d, d), lambda i: (0, 0)),
            pl.BlockSpec((1, d), lambda i: (0, 0)),
        ],
        out_specs=pl.BlockSpec((block_n, d), lambda i: (i, 0)),
        out_shape=jax.ShapeDtypeStruct((n, d), jnp.float32),
    )(x, agg, w3, b3)


def _sc_gather_sum(x, idx_c, idx_n, n_win, base_win, window):
    d = x.shape[1]
    num_workers = _NC * _NS
    slots = ((-(-n_win // num_workers) + 2) // 2) * 2
    mesh = plsc.VectorSubcoreMesh(core_axis_name="core", subcore_axis_name="subcore")

    @functools.partial(
        pl.kernel,
        out_type=jax.ShapeDtypeStruct((n_win * window, d), jnp.float32),
        mesh=mesh,
        scratch_types=[
            pltpu.VMEM((window, d), jnp.float32),
            pltpu.VMEM((window, d), jnp.float32),
            pltpu.VMEM((window, d), jnp.float32),
            pltpu.VMEM((window, d), jnp.float32),
            pltpu.VMEM((1, window), jnp.int32),
            pltpu.VMEM((1, window), jnp.int32),
            pltpu.VMEM((1, window), jnp.int32),
            pltpu.VMEM((1, window), jnp.int32),
            pltpu.SemaphoreType.DMA,
            pltpu.SemaphoreType.DMA,
            pltpu.SemaphoreType.DMA,
            pltpu.SemaphoreType.DMA,
            pltpu.SemaphoreType.DMA,
            pltpu.SemaphoreType.DMA,
        ],
    )
    def gather_kernel(x_hbm, ic_hbm, in_hbm, s_hbm,
                      ce0, ce1, ne0, ne1, ic0, ic1, in0, in1,
                      si0, si1, sg0, sg1, so0, so1):
        cid = jax.lax.axis_index("core")
        sid = jax.lax.axis_index("subcore")
        wid = sid * _NC + cid
        ce = (ce0, ce1)
        ne = (ne0, ne1)
        icb = (ic0, ic1)
        inb = (in0, in1)
        si = (si0, si1)
        sg = (sg0, sg1)
        so = (so0, so1)

        pltpu.async_copy(
            ic_hbm.at[:, pl.ds((base_win + wid) * window, window)], icb[0], si[0]
        )
        pltpu.async_copy(
            in_hbm.at[:, pl.ds((base_win + wid) * window, window)], inb[0], si[0]
        )

        @pl.loop(0, slots // 2)
        def _outer(j):
            for p in range(2):
                k = j * 2 + p
                q = 1 - p
                w = wid + k * num_workers
                w_prev = w - num_workers
                w_next = w + num_workers

                @pl.when(w < n_win)
                def _():
                    @pl.when(k >= 2)
                    def _():
                        pltpu.make_async_copy(
                            ce[p], s_hbm.at[pl.ds(0, window)], so[p]
                        ).wait()

                    pltpu.make_async_copy(
                        ic_hbm.at[:, pl.ds(0, window)], icb[p], si[p]
                    ).wait()
                    pltpu.make_async_copy(
                        in_hbm.at[:, pl.ds(0, window)], inb[p], si[p]
                    ).wait()
                    pltpu.async_copy(x_hbm.at[icb[p].at[0]], ce[p], sg[p])
                    pltpu.async_copy(x_hbm.at[inb[p].at[0]], ne[p], sg[p])

                @pl.when((k >= 1) & (w_prev < n_win))
                def _():
                    pltpu.make_async_copy(
                        x_hbm.at[pl.ds(0, window)], ce[q], sg[q]
                    ).wait()
                    pltpu.make_async_copy(
                        x_hbm.at[pl.ds(0, window)], ne[q], sg[q]
                    ).wait()

                @pl.when(w_next < n_win)
                def _():
                    pltpu.async_copy(
                        ic_hbm.at[:, pl.ds((base_win + w_next) * window, window)],
                        icb[q], si[q],
                    )
                    pltpu.async_copy(
                        in_hbm.at[:, pl.ds((base_win + w_next) * window, window)],
                        inb[q], si[q],
                    )

                @pl.when((k >= 1) & (w_prev < n_win))
                def _():
                    @pl.loop(0, window)
                    def _row(r):
                        @pl.loop(0, d, step=_LANES)
                        def _col(c):
                            slc = (pl.ds(r, 1), pl.ds(c, _LANES))
                            ce[q].at[*slc][...] = (
                                ce[q].at[*slc][...] + ne[q].at[*slc][...]
                            )

                    pltpu.async_copy(
                        ce[q], s_hbm.at[pl.ds(w_prev * window, window)], so[q]
                    )

        pltpu.make_async_copy(ce0, s_hbm.at[pl.ds(0, window)], so0).wait()
        pltpu.make_async_copy(ce1, s_hbm.at[pl.ds(0, window)], so1).wait()

    return gather_kernel(x, idx_c, idx_n)


def _sc_scatter_agg(x, thetas, idx_c, idx_n, window):
    d = x.shape[1]
    n = x.shape[0]
    chunk_rows = 80
    num_chunks = n // chunk_rows
    chunks_per_tile = -(-num_chunks // _NS)
    mesh = plsc.VectorSubcoreMesh(core_axis_name="core", subcore_axis_name="subcore")

    num_workers = _NC * _NS
    chunk_wins = [t.shape[0] // window for t in thetas]
    chunk_bases = [sum(chunk_wins[:i]) for i in range(len(thetas))]

    @functools.partial(
        pl.kernel,
        out_type=jax.ShapeDtypeStruct((_NC, n, d), jnp.float32),
        mesh=mesh,
        scratch_types=[
            pltpu.VMEM((window, d), jnp.float32),
            pltpu.VMEM((window, d), jnp.float32),
            pltpu.VMEM((window, d), jnp.float32),
            pltpu.VMEM((1, window), jnp.int32),
            pltpu.VMEM((1, window), jnp.int32),
            pltpu.VMEM((1, window), jnp.int32),
            pltpu.VMEM((1, window), jnp.int32),
            pltpu.VMEM_SHARED((n, d), jnp.float32),
            pltpu.SemaphoreType.DMA,
            pltpu.SemaphoreType.DMA,
            pltpu.SemaphoreType.DMA,
            pltpu.SemaphoreType.DMA,
            pltpu.SemaphoreType.DMA,
            pltpu.SemaphoreType.DMA,
            pltpu.SemaphoreType.DMA,
            pltpu.SemaphoreType.DMA,
            pltpu.SemaphoreType.DMA,
        ],
    )
    def scatter_kernel(x_hbm, *rest):
        th_hbms = rest[: len(thetas)]
        (ic_hbm, in_hbm, agg_hbm,
         ne0, ne1, th_buf, ic0, ic1, in0, in1, shared,
         si0, si1, sic0, sic1, sg0, sg1, st, so0, so1) = rest[len(thetas):]
        cid = jax.lax.axis_index("core")
        sid = jax.lax.axis_index("subcore")
        wid = sid * _NC + cid
        ne = (ne0, ne1)
        icb = (ic0, ic1)
        inb = (in0, in1)
        si = (si0, si1)
        sic = (sic0, sic1)
        sg = (sg0, sg1)
        so = (so0, so1)

        @pl.loop(0, chunk_rows)
        def _zr(r):
            @pl.loop(0, d, step=_LANES)
            def _zc(c):
                ne0.at[pl.ds(r, 1), pl.ds(c, _LANES)][...] = jnp.zeros(
                    (1, _LANES), jnp.float32
                )

        @pl.loop(0, chunks_per_tile)
        def _z(k):
            j = sid + k * _NS

            @pl.when(j < num_chunks)
            def _():
                pltpu.sync_copy(
                    ne0.at[pl.ds(0, chunk_rows)],
                    shared.at[pl.ds(j * chunk_rows, chunk_rows)],
                )

        plsc.subcore_barrier()

        for ci in range(len(thetas)):
            th_hbm = th_hbms[ci]
            n_win = chunk_wins[ci]
            base_win = chunk_bases[ci]
            slots = ((-(-n_win // num_workers) + 2) // 2) * 2

            pltpu.async_copy(
                in_hbm.at[:, pl.ds((base_win + wid) * window, window)],
                inb[0], si[0],
            )

            @pl.loop(0, slots // 2)
            def _outer(j):
                for p in range(2):
                    k = j * 2 + p
                    q = 1 - p
                    w = wid + k * num_workers
                    w_prev = w - num_workers
                    w_next = w + num_workers
                    base = w * window

                    @pl.when(w < n_win)
                    def _():
                        @pl.when(k >= 2)
                        def _():
                            pltpu.make_async_copy(
                                ne[p], shared.at[pl.ds(0, window)], so[p]
                            ).wait()

                        pltpu.make_async_copy(
                            in_hbm.at[:, pl.ds(0, window)], inb[p], si[p]
                        ).wait()
                        pltpu.async_copy(x_hbm.at[inb[p].at[0]], ne[p], sg[p])

                    @pl.when((k >= 1) & (w_prev < n_win))
                    def _():
                        pltpu.make_async_copy(
                            x_hbm.at[pl.ds(0, window)], ne[q], sg[q]
                        ).wait()

                    @pl.when(w_next < n_win)
                    def _():
                        pltpu.async_copy(
                            in_hbm.at[
                                :, pl.ds((base_win + w_next) * window, window)
                            ],
                            inb[q], si[q],
                        )

                    @pl.when((k >= 1) & (w_prev < n_win))
                    def _():
                        pltpu.make_async_copy(
                            th_hbm.at[pl.ds(0, window)], th_buf, st
                        ).wait()

                        @pl.loop(0, window)
                        def _row(r):
                            @pl.loop(0, d, step=_LANES)
                            def _col(c):
                                slc = (pl.ds(r, 1), pl.ds(c, _LANES))
                                ne[q].at[*slc][...] = (
                                    ne[q].at[*slc][...] * th_buf.at[*slc][...]
                                )

                    @pl.when(w < n_win)
                    def _():
                        pltpu.async_copy(
                            th_hbm.at[pl.ds(base, window)], th_buf, st
                        )

                    @pl.when((k >= 1) & (w_prev < n_win))
                    def _():
                        pltpu.make_async_copy(
                            ic_hbm.at[:, pl.ds(0, window)], icb[q], sic[q]
                        ).wait()
                        pltpu.async_copy(
                            ne[q], shared.at[icb[q].at[0]], so[q], add=True
                        )

                    @pl.when(w < n_win)
                    def _():
                        pltpu.async_copy(
                            ic_hbm.at[
                                :, pl.ds((base_win + w) * window, window)
                            ],
                            icb[p], sic[p],
                        )

            pltpu.make_async_copy(ne0, shared.at[pl.ds(0, window)], so0).wait()
            pltpu.make_async_copy(ne1, shared.at[pl.ds(0, window)], so1).wait()

        plsc.subcore_barrier()

        @pl.loop(0, chunks_per_tile)
        def _d(k):
            j = sid + k * _NS

            @pl.when(j < num_chunks)
            def _():
                off = j * chunk_rows
                pltpu.sync_copy(shared.at[pl.ds(off, chunk_rows)],
                                th_buf.at[pl.ds(0, chunk_rows)])
                pltpu.sync_copy(th_buf.at[pl.ds(0, chunk_rows)],
                                agg_hbm.at[cid, pl.ds(off, chunk_rows)])

    return scatter_kernel(x, *thetas, idx_c, idx_n)


def kernel(node_embeddings, edge_embeddings, edge_index_list,
           ln_gamma, ln_beta, W1, b1, W2, b2, W3, b3):
    n, d = node_embeddings.shape
    e = edge_embeddings.shape[0]
    h = W1.shape[1]

    idx = edge_index_list.astype(jnp.int32)
    idx_c = idx[0:1, :]
    idx_n = idx[1:2, :]

    window = 128
    n_chunks = 4
    block_e = 3200
    total_win = e // window
    wins_per_chunk = total_win // n_chunks

    x = _layer_norm_tc(
        node_embeddings, ln_gamma.reshape(1, d), ln_beta.reshape(1, d), block_n=400
    )

    w1b = W1.astype(jnp.bfloat16)
    w2b = W2.astype(jnp.bfloat16)
    thetas = []
    for c in range(n_chunks):
        s_c = _sc_gather_sum(x, idx_c, idx_n, wins_per_chunk,
                             c * wins_per_chunk, window)
        thetas.append(
            _mlp_tc(edge_embeddings, s_c, w1b, b1.reshape(1, h), w2b,
                    b2.reshape(1, d), block_e=block_e,
                    chunk_block0=c * wins_per_chunk * window // block_e)
        )
    agg = _sc_scatter_agg(x, thetas, idx_c, idx_n, window)
    out = _final_tc(x, agg, W3, b3.reshape(1, d), block_n=400)
    return out


# speedup vs baseline: 4.4285x; 1.0009x over previous
import functools

import jax
import jax.numpy as jnp
from jax.experimental import pallas as pl
from jax.experimental.pallas import tpu as pltpu
from jax.experimental.pallas import tpu_sc as plsc

_NC = 2
_NS = 16
_LANES = 16


def _silu(v):
    return v * jax.nn.sigmoid(v)


def _ln_body(x_ref, g_ref, b_ref, o_ref):
    x = x_ref[...]
    mu = jnp.mean(x, axis=1, keepdims=True)
    xc = x - mu
    var = jnp.mean(xc * xc, axis=1, keepdims=True)
    o_ref[...] = xc / jnp.sqrt(var + 1e-5) * g_ref[...] + b_ref[...]


def _mlp_body(e_ref, s_ref, w1_ref, b1_ref, w2_ref, b2_ref, o_ref):
    h = _silu(e_ref[...] + s_ref[...]).astype(jnp.bfloat16)
    h = jnp.dot(h, w1_ref[...], preferred_element_type=jnp.float32) + b1_ref[...]
    h = _silu(h).astype(jnp.bfloat16)
    o_ref[...] = (
        jnp.dot(h, w2_ref[...], preferred_element_type=jnp.float32) + b2_ref[...]
    )


def _out_body(x_ref, a_ref, w3_ref, b3_ref, o_ref):
    t = _silu(x_ref[...] + a_ref[0] + a_ref[1])
    o_ref[...] = (
        jnp.dot(t, w3_ref[...], preferred_element_type=jnp.float32) + b3_ref[...]
    )


def _layer_norm_tc(x, gamma, beta, block_n):
    n, d = x.shape
    grid = n // block_n
    return pl.pallas_call(
        _ln_body,
        grid=(grid,),
        in_specs=[
            pl.BlockSpec((block_n, d), lambda i: (i, 0)),
            pl.BlockSpec((1, d), lambda i: (0, 0)),
            pl.BlockSpec((1, d), lambda i: (0, 0)),
        ],
        out_specs=pl.BlockSpec((block_n, d), lambda i: (i, 0)),
        out_shape=jax.ShapeDtypeStruct((n, d), jnp.float32),
    )(x, gamma, beta)


def _mlp_tc(edge_emb, s, w1, b1, w2, b2, block_e, chunk_block0):
    ec, d = s.shape
    h = w1.shape[1]
    grid = ec // block_e
    return pl.pallas_call(
        _mlp_body,
        grid=(grid,),
        in_specs=[
            pl.BlockSpec((block_e, d), lambda i: (chunk_block0 + i, 0)),
            pl.BlockSpec((block_e, d), lambda i: (i, 0)),
            pl.BlockSpec((d, h), lambda i: (0, 0)),
            pl.BlockSpec((1, h), lambda i: (0, 0)),
            pl.BlockSpec((h, d), lambda i: (0, 0)),
            pl.BlockSpec((1, d), lambda i: (0, 0)),
        ],
        out_specs=pl.BlockSpec((block_e, d), lambda i: (i, 0)),
        out_shape=jax.ShapeDtypeStruct((ec, d), jnp.float32),
    )(edge_emb, s, w1, b1, w2, b2)


def _final_tc(x, agg, w3, b3, block_n):
    n, d = x.shape
    grid = n // block_n
    return pl.pallas_call(
        _out_body,
        grid=(grid,),
        in_specs=[
            pl.BlockSpec((block_n, d), lambda i: (i, 0)),
            pl.BlockSpec((2, block_n, d), lambda i: (0, i, 0)),
            pl.BlockSpec((d, d), lambda i: (0, 0)),
            pl.BlockSpec((1, d), lambda i: (0, 0)),
        ],
        out_specs=pl.BlockSpec((block_n, d), lambda i: (i, 0)),
        out_shape=jax.ShapeDtypeStruct((n, d), jnp.float32),
    )(x, agg, w3, b3)


def _sc_gather_sum(x, idx_c, idx_n, n_win, base_win, window):
    d = x.shape[1]
    num_workers = _NC * _NS
    slots = ((-(-n_win // num_workers) + 2) // 2) * 2
    mesh = plsc.VectorSubcoreMesh(core_axis_name="core", subcore_axis_name="subcore")

    @functools.partial(
        pl.kernel,
        out_type=jax.ShapeDtypeStruct((n_win * window, d), jnp.float32),
        mesh=mesh,
        scratch_types=[
            pltpu.VMEM((window, d), jnp.float32),
            pltpu.VMEM((window, d), jnp.float32),
            pltpu.VMEM((window, d), jnp.float32),
            pltpu.VMEM((window, d), jnp.float32),
            pltpu.VMEM((1, window), jnp.int32),
            pltpu.VMEM((1, window), jnp.int32),
            pltpu.VMEM((1, window), jnp.int32),
            pltpu.VMEM((1, window), jnp.int32),
            pltpu.SemaphoreType.DMA,
            pltpu.SemaphoreType.DMA,
            pltpu.SemaphoreType.DMA,
            pltpu.SemaphoreType.DMA,
            pltpu.SemaphoreType.DMA,
            pltpu.SemaphoreType.DMA,
        ],
    )
    def gather_kernel(x_hbm, ic_hbm, in_hbm, s_hbm,
                      ce0, ce1, ne0, ne1, ic0, ic1, in0, in1,
                      si0, si1, sg0, sg1, so0, so1):
        cid = jax.lax.axis_index("core")
        sid = jax.lax.axis_index("subcore")
        wid = sid * _NC + cid
        ce = (ce0, ce1)
        ne = (ne0, ne1)
        icb = (ic0, ic1)
        inb = (in0, in1)
        si = (si0, si1)
        sg = (sg0, sg1)
        so = (so0, so1)

        pltpu.async_copy(
            ic_hbm.at[:, pl.ds((base_win + wid) * window, window)], icb[0], si[0]
        )
        pltpu.async_copy(
            in_hbm.at[:, pl.ds((base_win + wid) * window, window)], inb[0], si[0]
        )

        @pl.loop(0, slots // 2)
        def _outer(j):
            for p in range(2):
                k = j * 2 + p
                q = 1 - p
                w = wid + k * num_workers
                w_prev = w - num_workers
                w_next = w + num_workers

                @pl.when(w < n_win)
                def _():
                    @pl.when(k >= 2)
                    def _():
                        pltpu.make_async_copy(
                            ce[p], s_hbm.at[pl.ds(0, window)], so[p]
                        ).wait()

                    pltpu.make_async_copy(
                        ic_hbm.at[:, pl.ds(0, window)], icb[p], si[p]
                    ).wait()
                    pltpu.make_async_copy(
                        in_hbm.at[:, pl.ds(0, window)], inb[p], si[p]
                    ).wait()
                    pltpu.async_copy(x_hbm.at[icb[p].at[0]], ce[p], sg[p])
                    pltpu.async_copy(x_hbm.at[inb[p].at[0]], ne[p], sg[p])

                @pl.when((k >= 1) & (w_prev < n_win))
                def _():
                    pltpu.make_async_copy(
                        x_hbm.at[pl.ds(0, window)], ce[q], sg[q]
                    ).wait()
                    pltpu.make_async_copy(
                        x_hbm.at[pl.ds(0, window)], ne[q], sg[q]
                    ).wait()

                @pl.when(w_next < n_win)
                def _():
                    pltpu.async_copy(
                        ic_hbm.at[:, pl.ds((base_win + w_next) * window, window)],
                        icb[q], si[q],
                    )
                    pltpu.async_copy(
                        in_hbm.at[:, pl.ds((base_win + w_next) * window, window)],
                        inb[q], si[q],
                    )

                @pl.when((k >= 1) & (w_prev < n_win))
                def _():
                    @pl.loop(0, window)
                    def _row(r):
                        @pl.loop(0, d, step=_LANES)
                        def _col(c):
                            slc = (pl.ds(r, 1), pl.ds(c, _LANES))
                            ce[q].at[*slc][...] = (
                                ce[q].at[*slc][...] + ne[q].at[*slc][...]
                            )

                    pltpu.async_copy(
                        ce[q], s_hbm.at[pl.ds(w_prev * window, window)], so[q]
                    )

        pltpu.make_async_copy(ce0, s_hbm.at[pl.ds(0, window)], so0).wait()
        pltpu.make_async_copy(ce1, s_hbm.at[pl.ds(0, window)], so1).wait()

    return gather_kernel(x, idx_c, idx_n)


def _sc_scatter_agg(x, thetas, idx_c, idx_n, window):
    d = x.shape[1]
    n = x.shape[0]
    chunk_rows = 80
    num_chunks = n // chunk_rows
    chunks_per_tile = -(-num_chunks // _NS)
    mesh = plsc.VectorSubcoreMesh(core_axis_name="core", subcore_axis_name="subcore")

    num_workers = _NC * _NS
    chunk_wins = [t.shape[0] // window for t in thetas]
    chunk_bases = [sum(chunk_wins[:i]) for i in range(len(thetas))]

    @functools.partial(
        pl.kernel,
        out_type=jax.ShapeDtypeStruct((_NC, n, d), jnp.float32),
        mesh=mesh,
        scratch_types=[
            pltpu.VMEM((window, d), jnp.float32),
            pltpu.VMEM((window, d), jnp.float32),
            pltpu.VMEM((window, d), jnp.float32),
            pltpu.VMEM((1, window), jnp.int32),
            pltpu.VMEM((1, window), jnp.int32),
            pltpu.VMEM((1, window), jnp.int32),
            pltpu.VMEM((1, window), jnp.int32),
            pltpu.VMEM_SHARED((n, d), jnp.float32),
            pltpu.SemaphoreType.DMA,
            pltpu.SemaphoreType.DMA,
            pltpu.SemaphoreType.DMA,
            pltpu.SemaphoreType.DMA,
            pltpu.SemaphoreType.DMA,
            pltpu.SemaphoreType.DMA,
            pltpu.SemaphoreType.DMA,
            pltpu.SemaphoreType.DMA,
            pltpu.SemaphoreType.DMA,
        ],
    )
    def scatter_kernel(x_hbm, *rest):
        th_hbms = rest[: len(thetas)]
        (ic_hbm, in_hbm, agg_hbm,
         ne0, ne1, th_buf, ic0, ic1, in0, in1, shared,
         si0, si1, sic0, sic1, sg0, sg1, st, so0, so1) = rest[len(thetas):]
        cid = jax.lax.axis_index("core")
        sid = jax.lax.axis_index("subcore")
        wid = sid * _NC + cid
        ne = (ne0, ne1)
        icb = (ic0, ic1)
        inb = (in0, in1)
        si = (si0, si1)
        sic = (sic0, sic1)
        sg = (sg0, sg1)
        so = (so0, so1)

        @pl.loop(0, chunk_rows)
        def _zr(r):
            @pl.loop(0, d, step=_LANES)
            def _zc(c):
                ne0.at[pl.ds(r, 1), pl.ds(c, _LANES)][...] = jnp.zeros(
                    (1, _LANES), jnp.float32
                )

        @pl.loop(0, chunks_per_tile)
        def _z(k):
            j = sid + k * _NS

            @pl.when(j < num_chunks)
            def _():
                pltpu.sync_copy(
                    ne0.at[pl.ds(0, chunk_rows)],
                    shared.at[pl.ds(j * chunk_rows, chunk_rows)],
                )

        plsc.subcore_barrier()

        for ci in range(len(thetas)):
            th_hbm = th_hbms[ci]
            n_win = chunk_wins[ci]
            base_win = chunk_bases[ci]
            slots = ((-(-n_win // num_workers) + 2) // 2) * 2

            pltpu.async_copy(
                in_hbm.at[:, pl.ds((base_win + wid) * window, window)],
                inb[0], si[0],
            )

            @pl.loop(0, slots // 2)
            def _outer(j):
                for p in range(2):
                    k = j * 2 + p
                    q = 1 - p
                    w = wid + k * num_workers
                    w_prev = w - num_workers
                    w_next = w + num_workers
                    base = w * window

                    @pl.when(w < n_win)
                    def _():
                        @pl.when(k >= 2)
                        def _():
                            pltpu.make_async_copy(
                                ne[p], shared.at[pl.ds(0, window)], so[p]
                            ).wait()

                        pltpu.make_async_copy(
                            in_hbm.at[:, pl.ds(0, window)], inb[p], si[p]
                        ).wait()
                        pltpu.async_copy(x_hbm.at[inb[p].at[0]], ne[p], sg[p])

                    @pl.when((k >= 1) & (w_prev < n_win))
                    def _():
                        pltpu.make_async_copy(
                            x_hbm.at[pl.ds(0, window)], ne[q], sg[q]
                        ).wait()

                    @pl.when(w_next < n_win)
                    def _():
                        pltpu.async_copy(
                            in_hbm.at[
                                :, pl.ds((base_win + w_next) * window, window)
                            ],
                            inb[q], si[q],
                        )

                    @pl.when((k >= 1) & (w_prev < n_win))
                    def _():
                        pltpu.make_async_copy(
                            th_hbm.at[pl.ds(0, window)], th_buf, st
                        ).wait()

                        @pl.loop(0, window)
                        def _row(r):
                            @pl.loop(0, d, step=_LANES)
                            def _col(c):
                                slc = (pl.ds(r, 1), pl.ds(c, _LANES))
                                ne[q].at[*slc][...] = (
                                    ne[q].at[*slc][...] * th_buf.at[*slc][...]
                                )

                    @pl.when(w < n_win)
                    def _():
                        pltpu.async_copy(
                            th_hbm.at[pl.ds(base, window)], th_buf, st
                        )

                    @pl.when((k >= 1) & (w_prev < n_win))
                    def _():
                        pltpu.make_async_copy(
                            ic_hbm.at[:, pl.ds(0, window)], icb[q], sic[q]
                        ).wait()
                        pltpu.async_copy(
                            ne[q], shared.at[icb[q].at[0]], so[q], add=True
                        )

                    @pl.when(w < n_win)
                    def _():
                        pltpu.async_copy(
                            ic_hbm.at[
                                :, pl.ds((base_win + w) * window, window)
                            ],
                            icb[p], sic[p],
                        )

            pltpu.make_async_copy(ne0, shared.at[pl.ds(0, window)], so0).wait()
            pltpu.make_async_copy(ne1, shared.at[pl.ds(0, window)], so1).wait()

        plsc.subcore_barrier()

        @pl.loop(0, chunks_per_tile)
        def _d(k):
            j = sid + k * _NS

            @pl.when(j < num_chunks)
            def _():
                off = j * chunk_rows
                pltpu.sync_copy(shared.at[pl.ds(off, chunk_rows)],
                                th_buf.at[pl.ds(0, chunk_rows)])
                pltpu.sync_copy(th_buf.at[pl.ds(0, chunk_rows)],
                                agg_hbm.at[cid, pl.ds(off, chunk_rows)])

    return scatter_kernel(x, *thetas, idx_c, idx_n)


def kernel(node_embeddings, edge_embeddings, edge_index_list,
           ln_gamma, ln_beta, W1, b1, W2, b2, W3, b3):
    n, d = node_embeddings.shape
    e = edge_embeddings.shape[0]
    h = W1.shape[1]

    idx = edge_index_list.astype(jnp.int32)
    idx_c = idx[0:1, :]
    idx_n = idx[1:2, :]

    window = 128
    block_e = 3200
    chunk_wins = [800, 800, 800, 100]

    x = _layer_norm_tc(
        node_embeddings, ln_gamma.reshape(1, d), ln_beta.reshape(1, d), block_n=400
    )

    w1b = W1.astype(jnp.bfloat16)
    w2b = W2.astype(jnp.bfloat16)
    thetas = []
    base = 0
    for wins in chunk_wins:
        s_c = _sc_gather_sum(x, idx_c, idx_n, wins, base, window)
        thetas.append(
            _mlp_tc(edge_embeddings, s_c, w1b, b1.reshape(1, h), w2b,
                    b2.reshape(1, d), block_e=block_e,
                    chunk_block0=base * window // block_e)
        )
        base += wins
    agg = _sc_scatter_agg(x, thetas, idx_c, idx_n, window)
    out = _final_tc(x, agg, W3, b3.reshape(1, d), block_n=400)
    return out
